# baseline (speedup 1.0000x reference)
import functools

import jax
import jax.numpy as jnp
from jax import lax
from jax.experimental import pallas as pl
from jax.experimental.pallas import tpu as pltpu
from jax.experimental.pallas import tpu_sc as plsc

_NC = 2
_NS = 16
_NW = _NC * _NS
_CH = 128


def _emb_lookup(idx, table, n_per_w, n_ch, d):
    mesh = plsc.VectorSubcoreMesh(core_axis_name="c", subcore_axis_name="s")

    @functools.partial(
        pl.kernel,
        mesh=mesh,
        out_type=jax.ShapeDtypeStruct((_NW * n_per_w, d), jnp.float32),
        scratch_types=[
            pltpu.VMEM((n_ch, _CH), jnp.int32),
            pltpu.VMEM((_CH, d), jnp.float32),
            pltpu.SemaphoreType.DMA,
        ],
        compiler_params=pltpu.CompilerParams(use_tc_tiling_on_sc=False),
    )
    def body(idx_hbm, tbl_hbm, out_hbm, idx_v, rows_v, gsem):
        wid = lax.axis_index("s") * _NC + lax.axis_index("c")
        base = wid * n_per_w
        pltpu.sync_copy(idx_hbm.at[wid], idx_v)

        def step(k, carry):
            pltpu.async_copy(tbl_hbm.at[idx_v.at[k]], rows_v, gsem).wait()
            pltpu.sync_copy(rows_v, out_hbm.at[pl.ds(base + k * _CH, _CH)])
            return carry

        lax.fori_loop(0, n_ch, step, 0)

    return body(idx, table)


def kernel(inputs, embeddings):
    b, s = inputs.shape
    v, d = embeddings.shape
    n = b * s
    n_per_w = n // _NW
    n_ch = n_per_w // _CH
    idx = inputs.reshape(_NW, n_ch, _CH)
    out = _emb_lookup(idx, embeddings, n_per_w, n_ch, d)
    return out.reshape(b, s, d)


# speedup vs baseline: 1.0385x; 1.0385x over previous
import functools

import jax
import jax.numpy as jnp
from jax import lax
from jax.experimental import pallas as pl
from jax.experimental.pallas import tpu as pltpu
from jax.experimental.pallas import tpu_sc as plsc

_NC = 2
_NS = 16
_NW = _NC * _NS
_CH = 128


def _emb_lookup(idx, table, n_per_w, n_ch, d):
    mesh = plsc.VectorSubcoreMesh(core_axis_name="c", subcore_axis_name="s")

    @functools.partial(
        pl.kernel,
        mesh=mesh,
        out_type=jax.ShapeDtypeStruct((_NW * n_per_w, d), jnp.float32),
        scratch_types=[
            pltpu.VMEM((n_ch, _CH), jnp.int32),
            pltpu.VMEM((2, _CH, d), jnp.float32),
            pltpu.SemaphoreType.DMA,
            pltpu.SemaphoreType.DMA,
            pltpu.SemaphoreType.DMA,
            pltpu.SemaphoreType.DMA,
        ],
        compiler_params=pltpu.CompilerParams(use_tc_tiling_on_sc=False),
    )
    def body(idx_hbm, tbl_hbm, out_hbm, idx_v, rows_v, g0, g1, o0, o1):
        wid = lax.axis_index("s") * _NC + lax.axis_index("c")
        base = wid * n_per_w
        pltpu.sync_copy(idx_hbm.at[wid], idx_v)
        gsems = (g0, g1)
        osems = (o0, o1)

        def start_gather(k, b, sem):
            pltpu.async_copy(tbl_hbm.at[idx_v.at[k]], rows_v.at[b], sem)

        def wait_gather(b, sem):
            pltpu.make_async_copy(
                tbl_hbm.at[pl.ds(0, _CH)], rows_v.at[b], sem).wait()

        def start_out(k, b, sem):
            pltpu.async_copy(
                rows_v.at[b], out_hbm.at[pl.ds(base + k * _CH, _CH)], sem)

        def wait_out(b, sem):
            pltpu.make_async_copy(
                rows_v.at[b], out_hbm.at[pl.ds(base, _CH)], sem).wait()

        start_gather(0, 0, gsems[0])

        def step(j, carry):
            for b in range(2):
                k = 2 * j + b
                o = 1 - b

                @pl.when(k >= 1)
                def _():
                    wait_out(o, osems[o])

                @pl.when(k + 1 < n_ch)
                def _():
                    start_gather(k + 1, o, gsems[o])

                wait_gather(b, gsems[b])
                start_out(k, b, osems[b])
            return carry

        lax.fori_loop(0, n_ch // 2, step, 0)
        wait_out((n_ch - 1) % 2, osems[(n_ch - 1) % 2])

    return body(idx, table)


def kernel(inputs, embeddings):
    b, s = inputs.shape
    v, d = embeddings.shape
    n = b * s
    n_per_w = n // _NW
    n_ch = n_per_w // _CH
    idx = inputs.reshape(_NW, n_ch, _CH)
    out = _emb_lookup(idx, embeddings, n_per_w, n_ch, d)
    return out.reshape(b, s, d)


# speedup vs baseline: 1.1453x; 1.1028x over previous
import functools

import jax
import jax.numpy as jnp
from jax import lax
from jax.experimental import pallas as pl
from jax.experimental.pallas import tpu as pltpu
from jax.experimental.pallas import tpu_sc as plsc

_NC = 2
_NS = 16
_NW = _NC * _NS
_CH = 128

_TBLK = 2048


def _transpose_tc(tbl_t, v, d):
    grid = (v + _TBLK - 1) // _TBLK

    def body(in_ref, out_ref):
        xt = in_ref[...].T
        out_ref[...] = jnp.concatenate([xt, jnp.zeros_like(xt)], axis=1)

    return pl.pallas_call(
        body,
        grid=(grid,),
        in_specs=[pl.BlockSpec((d, _TBLK), lambda i: (0, i))],
        out_specs=pl.BlockSpec((_TBLK, 128), lambda i: (i, 0)),
        out_shape=jax.ShapeDtypeStruct((v, 128), jnp.float32),
    )(tbl_t)


def _emb_lookup(idx, table, n_per_w, n_ch):
    mesh = plsc.VectorSubcoreMesh(core_axis_name="c", subcore_axis_name="s")

    @functools.partial(
        pl.kernel,
        mesh=mesh,
        out_type=jax.ShapeDtypeStruct((_NW * n_per_w, 128), jnp.float32),
        scratch_types=[
            pltpu.VMEM((n_ch, _CH), jnp.int32),
            pltpu.VMEM((2, _CH, 128), jnp.float32),
            pltpu.SemaphoreType.DMA,
            pltpu.SemaphoreType.DMA,
            pltpu.SemaphoreType.DMA,
            pltpu.SemaphoreType.DMA,
        ],
        compiler_params=pltpu.CompilerParams(use_tc_tiling_on_sc=True),
    )
    def body(idx_hbm, tbl_hbm, out_hbm, idx_v, rows_v, g0, g1, o0, o1):
        wid = lax.axis_index("s") * _NC + lax.axis_index("c")
        base = wid * n_per_w
        pltpu.sync_copy(idx_hbm.at[wid], idx_v)
        gsems = (g0, g1)
        osems = (o0, o1)

        def start_gather(k, b, sem):
            pltpu.async_copy(tbl_hbm.at[idx_v.at[k]], rows_v.at[b], sem)

        def wait_gather(b, sem):
            pltpu.make_async_copy(
                tbl_hbm.at[pl.ds(0, _CH)], rows_v.at[b], sem).wait()

        def start_out(k, b, sem):
            pltpu.async_copy(
                rows_v.at[b], out_hbm.at[pl.ds(base + k * _CH, _CH)], sem)

        def wait_out(b, sem):
            pltpu.make_async_copy(
                rows_v.at[b], out_hbm.at[pl.ds(base, _CH)], sem).wait()

        start_gather(0, 0, gsems[0])

        def step(j, carry):
            for b in range(2):
                k = 2 * j + b
                o = 1 - b

                @pl.when(k >= 1)
                def _():
                    wait_out(o, osems[o])

                @pl.when(k + 1 < n_ch)
                def _():
                    start_gather(k + 1, o, gsems[o])

                wait_gather(b, gsems[b])
                start_out(k, b, osems[b])
            return carry

        lax.fori_loop(0, n_ch // 2, step, 0)
        wait_out((n_ch - 1) % 2, osems[(n_ch - 1) % 2])

    return body(idx, table)


def kernel(inputs, embeddings):
    b, s = inputs.shape
    v, d = embeddings.shape
    n = b * s
    n_per_w = n // _NW
    n_ch = n_per_w // _CH
    idx = inputs.reshape(_NW, n_ch, _CH)
    table = _transpose_tc(embeddings.T, v, d)
    out = _emb_lookup(idx, table, n_per_w, n_ch)
    return out[:, :d].reshape(b, s, d)


# speedup vs baseline: 1.5924x; 1.3904x over previous
import functools

import jax
import jax.numpy as jnp
from jax import lax
from jax.experimental import pallas as pl
from jax.experimental.pallas import tpu as pltpu
from jax.experimental.pallas import tpu_sc as plsc

_NC = 2
_NS = 16
_NW = _NC * _NS
_CH = 128

_TBLK = 8192


def _transpose_tc(tbl_t, v, d):
    grid = (v + _TBLK - 1) // _TBLK

    def body(in_ref, out_ref):
        out_ref[:, :d] = in_ref[...].T

    return pl.pallas_call(
        body,
        grid=(grid,),
        in_specs=[pl.BlockSpec((d, _TBLK), lambda i: (0, i))],
        out_specs=pl.BlockSpec((_TBLK, 128), lambda i: (i, 0)),
        out_shape=jax.ShapeDtypeStruct((v, 128), jnp.float32),
    )(tbl_t)


def _emb_lookup(idx, table, n_per_w, n_ch):
    mesh = plsc.VectorSubcoreMesh(core_axis_name="c", subcore_axis_name="s")

    @functools.partial(
        pl.kernel,
        mesh=mesh,
        out_type=jax.ShapeDtypeStruct((_NW * n_per_w, 128), jnp.float32),
        scratch_types=[
            pltpu.VMEM((n_ch, _CH), jnp.int32),
            pltpu.VMEM((2, _CH, 128), jnp.float32),
            pltpu.SemaphoreType.DMA,
            pltpu.SemaphoreType.DMA,
            pltpu.SemaphoreType.DMA,
            pltpu.SemaphoreType.DMA,
        ],
        compiler_params=pltpu.CompilerParams(use_tc_tiling_on_sc=True),
    )
    def body(idx_hbm, tbl_hbm, out_hbm, idx_v, rows_v, g0, g1, o0, o1):
        wid = lax.axis_index("s") * _NC + lax.axis_index("c")
        base = wid * n_per_w
        pltpu.sync_copy(idx_hbm.at[wid], idx_v)
        gsems = (g0, g1)
        osems = (o0, o1)

        def start_gather(k, b, sem):
            pltpu.async_copy(tbl_hbm.at[idx_v.at[k]], rows_v.at[b], sem)

        def wait_gather(b, sem):
            pltpu.make_async_copy(
                tbl_hbm.at[pl.ds(0, _CH)], rows_v.at[b], sem).wait()

        def start_out(k, b, sem):
            pltpu.async_copy(
                rows_v.at[b], out_hbm.at[pl.ds(base + k * _CH, _CH)], sem)

        def wait_out(b, sem):
            pltpu.make_async_copy(
                rows_v.at[b], out_hbm.at[pl.ds(base, _CH)], sem).wait()

        start_gather(0, 0, gsems[0])

        def step(j, carry):
            for b in range(2):
                k = 2 * j + b
                o = 1 - b

                @pl.when(k >= 1)
                def _():
                    wait_out(o, osems[o])

                @pl.when(k + 1 < n_ch)
                def _():
                    start_gather(k + 1, o, gsems[o])

                wait_gather(b, gsems[b])
                start_out(k, b, osems[b])
            return carry

        lax.fori_loop(0, n_ch // 2, step, 0)
        wait_out((n_ch - 1) % 2, osems[(n_ch - 1) % 2])

    return body(idx, table)


def kernel(inputs, embeddings):
    b, s = inputs.shape
    v, d = embeddings.shape
    n = b * s
    n_per_w = n // _NW
    n_ch = n_per_w // _CH
    idx = inputs.reshape(_NW, n_ch, _CH)
    table = _transpose_tc(embeddings.T, v, d)
    out = _emb_lookup(idx, table, n_per_w, n_ch)
    return out[:, :d].reshape(b, s, d)
